# Initial kernel scaffold; baseline (speedup 1.0000x reference)
import jax
import jax.numpy as jnp
from jax.experimental import pallas as pl


def kernel(x, edge_index, W1, b1, W2, b2, W3, b3, Wa, ba):
    raise NotImplementedError("write your pallas kernel here")


# speedup vs baseline: 22.5993x; 22.5993x over previous
import functools

import jax
import jax.numpy as jnp
from jax import lax
from jax.experimental import pallas as pl
from jax.experimental.pallas import tpu as pltpu
from jax.experimental.pallas import tpu_sc as plsc

N = 10000
E = 320000
D = 128
DH = D // 2
DO3 = 384

NC = 2
NS = 16
NW = NC * NS
CHUNK = 128
CPW = 80
E_PAD = NW * CPW * CHUNK
CPS = 160
NP = 10240
RPS = NP // NS
RCHUNK = 128
NCOPY = RPS // RCHUNK

_mesh = plsc.VectorSubcoreMesh(core_axis_name="c", subcore_axis_name="s")


@functools.partial(
    pl.kernel,
    out_type=jax.ShapeDtypeStruct((NC * NP, 16), jnp.float32),
    mesh=_mesh,
    scratch_types=[
        pltpu.VMEM((CPW, CHUNK), jnp.int32),
        pltpu.VMEM((CHUNK, 16), jnp.float32),
        pltpu.VMEM((RCHUNK, 16), jnp.float32),
        pltpu.VMEM_SHARED((NP, 16), jnp.float32),
    ],
    compiler_params=pltpu.CompilerParams(use_tc_tiling_on_sc=False),
)
def _deg_kernel(dstp_hbm, ones_hbm, zeros_hbm, out_hbm, dst_v, ones_v, buf_v, acc_sh):
    cid = lax.axis_index("c")
    sid = lax.axis_index("s")
    w = cid * NS + sid

    pltpu.sync_copy(zeros_hbm, buf_v)
    for k in range(NCOPY):
        pltpu.sync_copy(buf_v, acc_sh.at[pl.ds(sid * RPS + k * RCHUNK, RCHUNK)])
    pltpu.sync_copy(ones_hbm, ones_v)
    pltpu.sync_copy(dstp_hbm.at[pl.ds(w * CPW, CPW)], dst_v)
    plsc.subcore_barrier()

    def body(j, carry):
        pltpu.sync_copy(ones_v, acc_sh.at[dst_v.at[j]], add=True)
        return carry

    lax.fori_loop(0, CPW, body, 0)
    plsc.subcore_barrier()

    for k in range(NCOPY):
        r0 = sid * RPS + k * RCHUNK
        pltpu.sync_copy(acc_sh.at[pl.ds(r0, RCHUNK)], buf_v)
        pltpu.sync_copy(buf_v, out_hbm.at[pl.ds(cid * NP + r0, RCHUNK)])


@functools.partial(
    pl.kernel,
    out_type=jax.ShapeDtypeStruct((NC * NP, DH), jnp.float32),
    mesh=_mesh,
    scratch_types=[
        pltpu.VMEM((CPS, CHUNK), jnp.int32),
        pltpu.VMEM((CPS, CHUNK), jnp.int32),
        pltpu.VMEM((CHUNK, DH), jnp.float32),
        pltpu.VMEM((RCHUNK, DH), jnp.float32),
        pltpu.VMEM_SHARED((NP, DH), jnp.float32),
        pltpu.SemaphoreType.DMA,
    ],
    compiler_params=pltpu.CompilerParams(use_tc_tiling_on_sc=False),
)
def _scatter_kernel(ysplit_hbm, srcp_hbm, dstp_hbm, zeros_hbm, out_hbm,
                    src_v, dst_v, rows_v, buf_v, z_sh, sem):
    cid = lax.axis_index("c")
    sid = lax.axis_index("s")

    pltpu.sync_copy(zeros_hbm, buf_v)
    for k in range(NCOPY):
        pltpu.sync_copy(buf_v, z_sh.at[pl.ds(sid * RPS + k * RCHUNK, RCHUNK)])
    pltpu.sync_copy(srcp_hbm.at[pl.ds((cid * NS + sid) * CPS, CPS)], src_v)
    pltpu.sync_copy(dstp_hbm.at[pl.ds(sid * CPS, CPS)], dst_v)
    plsc.subcore_barrier()

    def body(j, carry):
        pltpu.async_copy(ysplit_hbm.at[src_v.at[j]], rows_v, sem).wait()
        pltpu.sync_copy(rows_v, z_sh.at[dst_v.at[j]], add=True)
        return carry

    lax.fori_loop(0, CPS, body, 0)
    plsc.subcore_barrier()

    for k in range(NCOPY):
        r0 = sid * RPS + k * RCHUNK
        pltpu.sync_copy(z_sh.at[pl.ds(r0, RCHUNK)], buf_v)
        pltpu.sync_copy(buf_v, out_hbm.at[pl.ds(cid * NP + r0, RCHUNK)])


_BLK = 512
_GRID = NP // _BLK


def _finish_body(za_ref, zb_ref, ya_ref, yb_ref, dinv_ref, wca_ref, wcb_ref,
                 bc_ref, wa_ref, ba_ref, out_ref):
    agg_a = (za_ref[...] + ya_ref[...]) * dinv_ref[...]
    agg_b = (zb_ref[...] + yb_ref[...]) * dinv_ref[...]
    h = (jnp.dot(agg_a, wca_ref[...], preferred_element_type=jnp.float32)
         + jnp.dot(agg_b, wcb_ref[...], preferred_element_type=jnp.float32))
    h = h + bc_ref[...]
    h = jnp.where(h > 0, h, 0.01 * h)
    logits = jnp.dot(h, wa_ref[...], preferred_element_type=jnp.float32)
    logits = logits + ba_ref[...]
    m = jnp.max(logits, axis=-1, keepdims=True)
    e = jnp.exp(logits - m)
    a = e / jnp.sum(e, axis=-1, keepdims=True)
    out_ref[...] = (h[:, 0:D] * a[:, 0:1] + h[:, D:2 * D] * a[:, 1:2]
                    + h[:, 2 * D:3 * D] * a[:, 2:3])


_finish = pl.pallas_call(
    _finish_body,
    grid=(_GRID,),
    in_specs=[
        pl.BlockSpec((_BLK, DH), lambda i: (i, 0)),
        pl.BlockSpec((_BLK, DH), lambda i: (i + _GRID, 0)),
        pl.BlockSpec((_BLK, DH), lambda i: (i, 0)),
        pl.BlockSpec((_BLK, DH), lambda i: (i + _GRID, 0)),
        pl.BlockSpec((_BLK, 1), lambda i: (i, 0)),
        pl.BlockSpec((DH, DO3), lambda i: (0, 0)),
        pl.BlockSpec((DH, DO3), lambda i: (0, 0)),
        pl.BlockSpec((1, DO3), lambda i: (0, 0)),
        pl.BlockSpec((DO3, 3), lambda i: (0, 0)),
        pl.BlockSpec((1, 3), lambda i: (0, 0)),
    ],
    out_specs=pl.BlockSpec((_BLK, D), lambda i: (i, 0)),
    out_shape=jax.ShapeDtypeStruct((N, D), jnp.float32),
)


def kernel(x, edge_index, W1, b1, W2, b2, W3, b3, Wa, ba):
    src = edge_index[0]
    dst = edge_index[1]
    pad = jnp.full((E_PAD - E,), N, dtype=jnp.int32)
    srcp = jnp.concatenate([src, pad])
    dstp = jnp.concatenate([dst, pad]).reshape(NW * CPW, CHUNK)

    ones16 = jnp.ones((CHUNK, 16), jnp.float32)
    zeros16 = jnp.zeros((RCHUNK, 16), jnp.float32)
    dego = _deg_kernel(dstp, ones16, zeros16)
    deg = dego[:NP, 0] + dego[NP:, 0] + 1.0
    dinv = lax.rsqrt(deg)

    y = jnp.zeros((NP, D), jnp.float32).at[:N].set(x * dinv[:N, None])
    ysplit = jnp.concatenate([y[:, :DH], y[:, DH:]], axis=0)
    srcp2 = jnp.concatenate([srcp, srcp + NP]).reshape(NC * NS * CPS, CHUNK)
    dstp2 = dstp.reshape(NS * CPS, CHUNK)
    zeros64 = jnp.zeros((RCHUNK, DH), jnp.float32)
    z = _scatter_kernel(ysplit, srcp2, dstp2, zeros64)

    Wcat = jnp.concatenate([W1, W2, W3], axis=1)
    bcat = jnp.concatenate([b1, b2, b3]).reshape(1, DO3)
    return _finish(z, z, ysplit, ysplit, dinv[:, None], Wcat[:DH], Wcat[DH:],
                   bcat, Wa, ba.reshape(1, 3))


# speedup vs baseline: 29.5869x; 1.3092x over previous
import functools

import jax
import jax.numpy as jnp
from jax import lax
from jax.experimental import pallas as pl
from jax.experimental.pallas import tpu as pltpu
from jax.experimental.pallas import tpu_sc as plsc

N = 10000
E = 320000
D = 128
DH = D // 2
DO3 = 384

NC = 2
NS = 16
NW = NC * NS
CHUNK = 128
CPW = 80
E_PAD = NW * CPW * CHUNK
CPS = 160
NP = 10240
RPS = NP // NS
RCHUNK = 128
NCOPY = RPS // RCHUNK

_mesh = plsc.VectorSubcoreMesh(core_axis_name="c", subcore_axis_name="s")


@functools.partial(
    pl.kernel,
    out_type=jax.ShapeDtypeStruct((NC * NP, 16), jnp.float32),
    mesh=_mesh,
    scratch_types=[
        pltpu.VMEM((CPW, CHUNK), jnp.int32),
        pltpu.VMEM((CHUNK, 16), jnp.float32),
        pltpu.VMEM((RCHUNK, 16), jnp.float32),
        pltpu.VMEM_SHARED((NP, 16), jnp.float32),
    ],
    compiler_params=pltpu.CompilerParams(use_tc_tiling_on_sc=False),
)
def _deg_kernel(dstp_hbm, ones_hbm, zeros_hbm, out_hbm, dst_v, ones_v, buf_v, acc_sh):
    cid = lax.axis_index("c")
    sid = lax.axis_index("s")
    w = cid * NS + sid

    pltpu.sync_copy(zeros_hbm, buf_v)
    for k in range(NCOPY):
        pltpu.sync_copy(buf_v, acc_sh.at[pl.ds(sid * RPS + k * RCHUNK, RCHUNK)])
    pltpu.sync_copy(ones_hbm, ones_v)
    pltpu.sync_copy(dstp_hbm.at[pl.ds(w * CPW, CPW)], dst_v)
    plsc.subcore_barrier()

    def body(j, carry):
        pltpu.sync_copy(ones_v, acc_sh.at[dst_v.at[j]], add=True)
        return carry

    lax.fori_loop(0, CPW, body, 0)
    plsc.subcore_barrier()

    for k in range(NCOPY):
        r0 = sid * RPS + k * RCHUNK
        pltpu.sync_copy(acc_sh.at[pl.ds(r0, RCHUNK)], buf_v)
        pltpu.sync_copy(buf_v, out_hbm.at[pl.ds(cid * NP + r0, RCHUNK)])


@functools.partial(
    pl.kernel,
    out_type=jax.ShapeDtypeStruct((NC * NP, DH), jnp.float32),
    mesh=_mesh,
    scratch_types=[
        pltpu.VMEM((CPS, CHUNK), jnp.int32),
        pltpu.VMEM((CPS, CHUNK), jnp.int32),
        pltpu.VMEM((CHUNK, DH), jnp.float32),
        pltpu.VMEM((CHUNK, DH), jnp.float32),
        pltpu.VMEM((RCHUNK, DH), jnp.float32),
        pltpu.VMEM_SHARED((NP, DH), jnp.float32),
        pltpu.SemaphoreType.DMA,
        pltpu.SemaphoreType.DMA,
    ],
    compiler_params=pltpu.CompilerParams(use_tc_tiling_on_sc=False),
)
def _scatter_kernel(ysplit_hbm, srcp_hbm, dstp_hbm, zeros_hbm, out_hbm,
                    src_v, dst_v, rows0_v, rows1_v, buf_v, z_sh, sem0, sem1):
    cid = lax.axis_index("c")
    sid = lax.axis_index("s")

    pltpu.sync_copy(zeros_hbm, buf_v)
    for k in range(NCOPY):
        pltpu.sync_copy(buf_v, z_sh.at[pl.ds(sid * RPS + k * RCHUNK, RCHUNK)])
    pltpu.sync_copy(srcp_hbm.at[pl.ds(sid * CPS, CPS)], src_v)
    pltpu.sync_copy(dstp_hbm.at[pl.ds(sid * CPS, CPS)], dst_v)

    off = cid * NP

    def addoff(j, carry):
        for k in range(CHUNK // 16):
            src_v[j, pl.ds(k * 16, 16)] = src_v[j, pl.ds(k * 16, 16)] + off
        return carry

    lax.fori_loop(0, CPS, addoff, 0)
    plsc.subcore_barrier()

    bufs = ((rows0_v, sem0), (rows1_v, sem1))
    for b, (buf, sem) in enumerate(bufs):
        pltpu.async_copy(ysplit_hbm.at[src_v.at[b]], buf, sem)

    def body(p, carry):
        for b, (buf, sem) in enumerate(bufs):
            j = 2 * p + b
            pltpu.make_async_copy(ysplit_hbm.at[src_v.at[j]], buf, sem).wait()
            pltpu.sync_copy(buf, z_sh.at[dst_v.at[j]], add=True)
            nxt = j + 2

            @pl.when(nxt < CPS)
            def _():
                pltpu.async_copy(ysplit_hbm.at[src_v.at[nxt]], buf, sem)

        return carry

    lax.fori_loop(0, CPS // 2, body, 0)
    plsc.subcore_barrier()

    for k in range(NCOPY):
        r0 = sid * RPS + k * RCHUNK
        pltpu.sync_copy(z_sh.at[pl.ds(r0, RCHUNK)], buf_v)
        pltpu.sync_copy(buf_v, out_hbm.at[pl.ds(cid * NP + r0, RCHUNK)])


_BLK = 512
_GRID = NP // _BLK


def _finish_body(za_ref, zb_ref, ya_ref, yb_ref, dinv_ref, wca_ref, wcb_ref,
                 bc_ref, wa_ref, ba_ref, out_ref):
    agg_a = (za_ref[...] + ya_ref[...]) * dinv_ref[...]
    agg_b = (zb_ref[...] + yb_ref[...]) * dinv_ref[...]
    h = (jnp.dot(agg_a, wca_ref[...], preferred_element_type=jnp.float32)
         + jnp.dot(agg_b, wcb_ref[...], preferred_element_type=jnp.float32))
    h = h + bc_ref[...]
    h = jnp.where(h > 0, h, 0.01 * h)
    logits = jnp.dot(h, wa_ref[...], preferred_element_type=jnp.float32)
    logits = logits + ba_ref[...]
    m = jnp.max(logits, axis=-1, keepdims=True)
    e = jnp.exp(logits - m)
    a = e / jnp.sum(e, axis=-1, keepdims=True)
    out_ref[...] = (h[:, 0:D] * a[:, 0:1] + h[:, D:2 * D] * a[:, 1:2]
                    + h[:, 2 * D:3 * D] * a[:, 2:3])


_finish = pl.pallas_call(
    _finish_body,
    grid=(_GRID,),
    in_specs=[
        pl.BlockSpec((_BLK, DH), lambda i: (i, 0)),
        pl.BlockSpec((_BLK, DH), lambda i: (i + _GRID, 0)),
        pl.BlockSpec((_BLK, DH), lambda i: (i, 0)),
        pl.BlockSpec((_BLK, DH), lambda i: (i + _GRID, 0)),
        pl.BlockSpec((_BLK, 1), lambda i: (i, 0)),
        pl.BlockSpec((DH, DO3), lambda i: (0, 0)),
        pl.BlockSpec((DH, DO3), lambda i: (0, 0)),
        pl.BlockSpec((1, DO3), lambda i: (0, 0)),
        pl.BlockSpec((DO3, 3), lambda i: (0, 0)),
        pl.BlockSpec((1, 3), lambda i: (0, 0)),
    ],
    out_specs=pl.BlockSpec((_BLK, D), lambda i: (i, 0)),
    out_shape=jax.ShapeDtypeStruct((N, D), jnp.float32),
)


def kernel(x, edge_index, W1, b1, W2, b2, W3, b3, Wa, ba):
    src = edge_index[0]
    dst = edge_index[1]
    pad = jnp.full((E_PAD - E,), N, dtype=jnp.int32)
    srcp = jnp.concatenate([src, pad])
    dstp = jnp.concatenate([dst, pad]).reshape(NW * CPW, CHUNK)

    ones16 = jnp.ones((CHUNK, 16), jnp.float32)
    zeros16 = jnp.zeros((RCHUNK, 16), jnp.float32)
    dego = _deg_kernel(dstp, ones16, zeros16)
    deg = dego[:NP, 0] + dego[NP:, 0] + 1.0
    dinv = lax.rsqrt(deg)

    xd = x * dinv[:N, None]
    ysplit = (jnp.zeros((NC * NP, DH), jnp.float32)
              .at[:N].set(xd[:, :DH]).at[NP:NP + N].set(xd[:, DH:]))
    srcp2 = srcp.reshape(NS * CPS, CHUNK)
    dstp2 = dstp.reshape(NS * CPS, CHUNK)
    zeros64 = jnp.zeros((RCHUNK, DH), jnp.float32)
    z = _scatter_kernel(ysplit, srcp2, dstp2, zeros64)

    Wcat = jnp.concatenate([W1, W2, W3], axis=1)
    bcat = jnp.concatenate([b1, b2, b3]).reshape(1, DO3)
    return _finish(z, z, ysplit, ysplit, dinv[:, None], Wcat[:DH], Wcat[DH:],
                   bcat, Wa, ba.reshape(1, 3))


# speedup vs baseline: 29.5981x; 1.0004x over previous
import functools

import jax
import jax.numpy as jnp
from jax import lax
from jax.experimental import pallas as pl
from jax.experimental.pallas import tpu as pltpu
from jax.experimental.pallas import tpu_sc as plsc

N = 10000
E = 320000
D = 128
DH = D // 2
DO3 = 384

NC = 2
NS = 16
NW = NC * NS
CHUNK = 128
CPW = 80
E_PAD = NW * CPW * CHUNK
CPS = 160
NP = 10240
RPS = NP // NS
RCHUNK = 128
NCOPY = RPS // RCHUNK

_mesh = plsc.VectorSubcoreMesh(core_axis_name="c", subcore_axis_name="s")


@functools.partial(
    pl.kernel,
    out_type=jax.ShapeDtypeStruct((NC * NP, 16), jnp.float32),
    mesh=_mesh,
    scratch_types=[
        pltpu.VMEM((CPW, CHUNK), jnp.int32),
        pltpu.VMEM((CHUNK, 16), jnp.float32),
        pltpu.VMEM((RCHUNK, 16), jnp.float32),
        pltpu.VMEM_SHARED((NP, 16), jnp.float32),
    ],
    compiler_params=pltpu.CompilerParams(use_tc_tiling_on_sc=False),
)
def _deg_kernel(dstp_hbm, ones_hbm, zeros_hbm, out_hbm, dst_v, ones_v, buf_v, acc_sh):
    cid = lax.axis_index("c")
    sid = lax.axis_index("s")
    w = cid * NS + sid

    pltpu.sync_copy(zeros_hbm, buf_v)
    for k in range(NCOPY):
        pltpu.sync_copy(buf_v, acc_sh.at[pl.ds(sid * RPS + k * RCHUNK, RCHUNK)])
    pltpu.sync_copy(ones_hbm, ones_v)
    pltpu.sync_copy(dstp_hbm.at[pl.ds(w * CPW, CPW)], dst_v)
    plsc.subcore_barrier()

    def body(j, carry):
        pltpu.sync_copy(ones_v, acc_sh.at[dst_v.at[j]], add=True)
        return carry

    lax.fori_loop(0, CPW, body, 0)
    plsc.subcore_barrier()

    for k in range(NCOPY):
        r0 = sid * RPS + k * RCHUNK
        pltpu.sync_copy(acc_sh.at[pl.ds(r0, RCHUNK)], buf_v)
        pltpu.sync_copy(buf_v, out_hbm.at[pl.ds(cid * NP + r0, RCHUNK)])


@functools.partial(
    pl.kernel,
    out_type=jax.ShapeDtypeStruct((NC * NP, DH), jnp.float32),
    mesh=_mesh,
    scratch_types=[
        pltpu.VMEM((CPS, CHUNK), jnp.int32),
        pltpu.VMEM((CPS, CHUNK), jnp.int32),
        pltpu.VMEM((CHUNK, DH), jnp.float32),
        pltpu.VMEM((CHUNK, DH), jnp.float32),
        pltpu.VMEM((CHUNK, DH), jnp.float32),
        pltpu.VMEM((CHUNK, DH), jnp.float32),
        pltpu.VMEM((RCHUNK, DH), jnp.float32),
        pltpu.VMEM_SHARED((NP, DH), jnp.float32),
        pltpu.SemaphoreType.DMA,
        pltpu.SemaphoreType.DMA,
        pltpu.SemaphoreType.DMA,
        pltpu.SemaphoreType.DMA,
        pltpu.SemaphoreType.DMA,
        pltpu.SemaphoreType.DMA,
        pltpu.SemaphoreType.DMA,
        pltpu.SemaphoreType.DMA,
    ],
    compiler_params=pltpu.CompilerParams(use_tc_tiling_on_sc=False),
)
def _scatter_kernel(ysplit_hbm, srcp_hbm, dstp_hbm, zeros_hbm, out_hbm,
                    src_v, dst_v, rows0_v, rows1_v, rows2_v, rows3_v, buf_v,
                    z_sh, gs0, gs1, gs2, gs3, ss0, ss1, ss2, ss3):
    cid = lax.axis_index("c")
    sid = lax.axis_index("s")

    pltpu.sync_copy(zeros_hbm, buf_v)
    for k in range(NCOPY):
        pltpu.sync_copy(buf_v, z_sh.at[pl.ds(sid * RPS + k * RCHUNK, RCHUNK)])
    pltpu.sync_copy(srcp_hbm.at[pl.ds(sid * CPS, CPS)], src_v)
    pltpu.sync_copy(dstp_hbm.at[pl.ds(sid * CPS, CPS)], dst_v)

    off = cid * NP

    def addoff(j, carry):
        for k in range(CHUNK // 16):
            src_v[j, pl.ds(k * 16, 16)] = src_v[j, pl.ds(k * 16, 16)] + off
        return carry

    lax.fori_loop(0, CPS, addoff, 0)
    plsc.subcore_barrier()

    bufs = ((rows0_v, gs0, ss0), (rows1_v, gs1, ss1),
            (rows2_v, gs2, ss2), (rows3_v, gs3, ss3))
    NBUF = len(bufs)
    for b, (buf, gsem, _) in enumerate(bufs):
        pltpu.async_copy(ysplit_hbm.at[src_v.at[b]], buf, gsem)

    def body(p, carry):
        base = NBUF * p
        for b, (buf, gsem, ssem) in enumerate(bufs):
            j = base + b
            pltpu.make_async_copy(ysplit_hbm.at[src_v.at[j]], buf, gsem).wait()
            pltpu.async_copy(buf, z_sh.at[dst_v.at[j]], ssem, add=True)
        for b, (buf, gsem, ssem) in enumerate(bufs):
            j = base + b
            nxt = j + NBUF
            pltpu.make_async_copy(buf, z_sh.at[dst_v.at[j]], ssem).wait()

            @pl.when(nxt < CPS)
            def _():
                pltpu.async_copy(ysplit_hbm.at[src_v.at[nxt]], buf, gsem)

        return carry

    lax.fori_loop(0, CPS // NBUF, body, 0)
    plsc.subcore_barrier()

    for k in range(NCOPY):
        r0 = sid * RPS + k * RCHUNK
        pltpu.sync_copy(z_sh.at[pl.ds(r0, RCHUNK)], buf_v)
        pltpu.sync_copy(buf_v, out_hbm.at[pl.ds(cid * NP + r0, RCHUNK)])


_BLK = 512
_GRID = NP // _BLK


def _finish_body(za_ref, zb_ref, ya_ref, yb_ref, dinv_ref, wca_ref, wcb_ref,
                 bc_ref, wa_ref, ba_ref, out_ref):
    agg_a = (za_ref[...] + ya_ref[...]) * dinv_ref[...]
    agg_b = (zb_ref[...] + yb_ref[...]) * dinv_ref[...]
    h = (jnp.dot(agg_a, wca_ref[...], preferred_element_type=jnp.float32)
         + jnp.dot(agg_b, wcb_ref[...], preferred_element_type=jnp.float32))
    h = h + bc_ref[...]
    h = jnp.where(h > 0, h, 0.01 * h)
    logits = jnp.dot(h, wa_ref[...], preferred_element_type=jnp.float32)
    logits = logits + ba_ref[...]
    m = jnp.max(logits, axis=-1, keepdims=True)
    e = jnp.exp(logits - m)
    a = e / jnp.sum(e, axis=-1, keepdims=True)
    out_ref[...] = (h[:, 0:D] * a[:, 0:1] + h[:, D:2 * D] * a[:, 1:2]
                    + h[:, 2 * D:3 * D] * a[:, 2:3])


_finish = pl.pallas_call(
    _finish_body,
    grid=(_GRID,),
    in_specs=[
        pl.BlockSpec((_BLK, DH), lambda i: (i, 0)),
        pl.BlockSpec((_BLK, DH), lambda i: (i + _GRID, 0)),
        pl.BlockSpec((_BLK, DH), lambda i: (i, 0)),
        pl.BlockSpec((_BLK, DH), lambda i: (i + _GRID, 0)),
        pl.BlockSpec((_BLK, 1), lambda i: (i, 0)),
        pl.BlockSpec((DH, DO3), lambda i: (0, 0)),
        pl.BlockSpec((DH, DO3), lambda i: (0, 0)),
        pl.BlockSpec((1, DO3), lambda i: (0, 0)),
        pl.BlockSpec((DO3, 3), lambda i: (0, 0)),
        pl.BlockSpec((1, 3), lambda i: (0, 0)),
    ],
    out_specs=pl.BlockSpec((_BLK, D), lambda i: (i, 0)),
    out_shape=jax.ShapeDtypeStruct((N, D), jnp.float32),
)


def kernel(x, edge_index, W1, b1, W2, b2, W3, b3, Wa, ba):
    src = edge_index[0]
    dst = edge_index[1]
    pad = jnp.full((E_PAD - E,), N, dtype=jnp.int32)
    srcp = jnp.concatenate([src, pad])
    dstp = jnp.concatenate([dst, pad]).reshape(NW * CPW, CHUNK)

    ones16 = jnp.ones((CHUNK, 16), jnp.float32)
    zeros16 = jnp.zeros((RCHUNK, 16), jnp.float32)
    dego = _deg_kernel(dstp, ones16, zeros16)
    deg = dego[:NP, 0] + dego[NP:, 0] + 1.0
    dinv = lax.rsqrt(deg)

    xd = x * dinv[:N, None]
    ysplit = (jnp.zeros((NC * NP, DH), jnp.float32)
              .at[:N].set(xd[:, :DH]).at[NP:NP + N].set(xd[:, DH:]))
    srcp2 = srcp.reshape(NS * CPS, CHUNK)
    dstp2 = dstp.reshape(NS * CPS, CHUNK)
    zeros64 = jnp.zeros((RCHUNK, DH), jnp.float32)
    z = _scatter_kernel(ysplit, srcp2, dstp2, zeros64)

    Wcat = jnp.concatenate([W1, W2, W3], axis=1)
    bcat = jnp.concatenate([b1, b2, b3]).reshape(1, DO3)
    return _finish(z, z, ysplit, ysplit, dinv[:, None], Wcat[:DH], Wcat[DH:],
                   bcat, Wa, ba.reshape(1, 3))


# speedup vs baseline: 34.4380x; 1.1635x over previous
import functools

import jax
import jax.numpy as jnp
from jax import lax
from jax.experimental import pallas as pl
from jax.experimental.pallas import tpu as pltpu
from jax.experimental.pallas import tpu_sc as plsc

N = 10000
E = 320000
D = 128
DH = D // 2
DO3 = 384

NC = 2
NS = 16
NW = NC * NS
CHUNK = 128
CPW = 80
E_PAD = NW * CPW * CHUNK
CPS = 160
NP = 10240
RPS = NP // NS
RCHUNK = 128
NCOPY = RPS // RCHUNK

_mesh = plsc.VectorSubcoreMesh(core_axis_name="c", subcore_axis_name="s")


@functools.partial(
    pl.kernel,
    out_type=jax.ShapeDtypeStruct((NC * NP, 16), jnp.float32),
    mesh=_mesh,
    scratch_types=[
        pltpu.VMEM((CPW, CHUNK), jnp.int32),
        pltpu.VMEM((CHUNK, 16), jnp.float32),
        pltpu.VMEM((RCHUNK, 16), jnp.float32),
        pltpu.VMEM_SHARED((NP, 16), jnp.float32),
    ],
    compiler_params=pltpu.CompilerParams(use_tc_tiling_on_sc=False),
)
def _deg_kernel(dstp_hbm, ones_hbm, zeros_hbm, out_hbm, dst_v, ones_v, buf_v, acc_sh):
    cid = lax.axis_index("c")
    sid = lax.axis_index("s")
    w = cid * NS + sid

    pltpu.sync_copy(zeros_hbm, buf_v)
    for k in range(NCOPY):
        pltpu.sync_copy(buf_v, acc_sh.at[pl.ds(sid * RPS + k * RCHUNK, RCHUNK)])
    pltpu.sync_copy(ones_hbm, ones_v)
    pltpu.sync_copy(dstp_hbm.at[pl.ds(w * CPW, CPW)], dst_v)
    plsc.subcore_barrier()

    def body(j, carry):
        pltpu.sync_copy(ones_v, acc_sh.at[dst_v.at[j]], add=True)
        return carry

    lax.fori_loop(0, CPW, body, 0)
    plsc.subcore_barrier()

    for k in range(NCOPY):
        r0 = sid * RPS + k * RCHUNK
        pltpu.sync_copy(acc_sh.at[pl.ds(r0, RCHUNK)], buf_v)
        pltpu.sync_copy(buf_v, out_hbm.at[pl.ds(cid * NP + r0, RCHUNK)])


@functools.partial(
    pl.kernel,
    out_type=jax.ShapeDtypeStruct((NC * NP, DH), jnp.float32),
    mesh=_mesh,
    scratch_types=[
        pltpu.VMEM((CPS // 2, CHUNK), jnp.int32),
        pltpu.VMEM((CPS // 2, CHUNK), jnp.int32),
        pltpu.VMEM((CHUNK, DH // 2), jnp.int32),
        pltpu.VMEM((CHUNK, DH // 2), jnp.int32),
        pltpu.VMEM((CHUNK, DH), jnp.float32),
        pltpu.VMEM((CHUNK, DH), jnp.float32),
        pltpu.VMEM((RCHUNK, DH), jnp.float32),
        pltpu.VMEM((RCHUNK, DH // 2), jnp.int32),
        pltpu.VMEM_SHARED((NP, DH // 2), jnp.int32),
        pltpu.VMEM_SHARED((NP, DH), jnp.float32),
        pltpu.SemaphoreType.DMA,
        pltpu.SemaphoreType.DMA,
        pltpu.SemaphoreType.DMA,
        pltpu.SemaphoreType.DMA,
    ],
    compiler_params=pltpu.CompilerParams(use_tc_tiling_on_sc=False),
)
def _scatter_kernel(ysplit_hbm, srcp_hbm, dstp_hbm, zeros_hbm, out_hbm,
                    src_v, dst_v, g0_v, g1_v, f0_v, f1_v, buf_v, tstage_v,
                    tbl_sh, z_sh, gs0, gs1, ss0, ss1):
    cid = lax.axis_index("c")
    sid = lax.axis_index("s")

    pltpu.sync_copy(zeros_hbm, buf_v)
    for k in range(NCOPY):
        r0 = sid * RPS + k * RCHUNK
        pltpu.sync_copy(ysplit_hbm.at[pl.ds(cid * NP + r0, RCHUNK)], tstage_v)
        pltpu.sync_copy(tstage_v, tbl_sh.at[pl.ds(r0, RCHUNK)])
        pltpu.sync_copy(buf_v, z_sh.at[pl.ds(r0, RCHUNK)])
    plsc.subcore_barrier()

    bufs = ((g0_v, f0_v, gs0, ss0), (g1_v, f1_v, gs1, ss1))
    HCH = CPS // 2

    def convert(gbuf, fbuf):
        def crow(r, carry):
            for g in range(DH // 32):
                v = gbuf[r, pl.ds(16 * g, 16)]
                a = lax.bitcast_convert_type(v << 16, jnp.float32)
                b = lax.bitcast_convert_type(v & jnp.int32(-65536), jnp.float32)
                fbuf[r, pl.ds(32 * g, 16)] = a
                fbuf[r, pl.ds(32 * g + 16, 16)] = b
            return carry

        lax.fori_loop(0, CHUNK, crow, 0)

    def body(p, carry):
        for b, (gbuf, fbuf, gsem, ssem) in enumerate(bufs):
            j = 2 * p + b
            pltpu.make_async_copy(tbl_sh.at[src_v.at[j]], gbuf, gsem).wait()

            @pl.when(j >= 2)
            def _():
                pltpu.make_async_copy(fbuf, z_sh.at[dst_v.at[j]], ssem).wait()

            convert(gbuf, fbuf)
            pltpu.async_copy(fbuf, z_sh.at[dst_v.at[j]], ssem, add=True)
            nxt = j + 2

            @pl.when(nxt < HCH)
            def _():
                pltpu.async_copy(tbl_sh.at[src_v.at[nxt]], gbuf, gsem)

        return carry

    for half in range(2):
        pltpu.sync_copy(srcp_hbm.at[pl.ds(sid * CPS + half * HCH, HCH)], src_v)
        pltpu.sync_copy(dstp_hbm.at[pl.ds(sid * CPS + half * HCH, HCH)], dst_v)
        for b, (gbuf, fbuf, gsem, _) in enumerate(bufs):
            pltpu.async_copy(tbl_sh.at[src_v.at[b]], gbuf, gsem)
        lax.fori_loop(0, HCH // 2, body, 0)
        for b, (gbuf, fbuf, gsem, ssem) in enumerate(bufs):
            pltpu.make_async_copy(fbuf, z_sh.at[dst_v.at[HCH - 2 + b]],
                                  ssem).wait()
    plsc.subcore_barrier()

    for k in range(NCOPY):
        r0 = sid * RPS + k * RCHUNK
        pltpu.sync_copy(z_sh.at[pl.ds(r0, RCHUNK)], buf_v)
        pltpu.sync_copy(buf_v, out_hbm.at[pl.ds(cid * NP + r0, RCHUNK)])


_BLK = 512
_GRID = NP // _BLK


def _finish_body(za_ref, zb_ref, ya_ref, yb_ref, dinv_ref, wca_ref, wcb_ref,
                 bc_ref, wa_ref, ba_ref, out_ref):
    agg_a = (za_ref[...] + ya_ref[...]) * dinv_ref[...]
    agg_b = (zb_ref[...] + yb_ref[...]) * dinv_ref[...]
    h = (jnp.dot(agg_a, wca_ref[...], preferred_element_type=jnp.float32)
         + jnp.dot(agg_b, wcb_ref[...], preferred_element_type=jnp.float32))
    h = h + bc_ref[...]
    h = jnp.where(h > 0, h, 0.01 * h)
    logits = jnp.dot(h, wa_ref[...], preferred_element_type=jnp.float32)
    logits = logits + ba_ref[...]
    m = jnp.max(logits, axis=-1, keepdims=True)
    e = jnp.exp(logits - m)
    a = e / jnp.sum(e, axis=-1, keepdims=True)
    out_ref[...] = (h[:, 0:D] * a[:, 0:1] + h[:, D:2 * D] * a[:, 1:2]
                    + h[:, 2 * D:3 * D] * a[:, 2:3])


_finish = pl.pallas_call(
    _finish_body,
    grid=(_GRID,),
    in_specs=[
        pl.BlockSpec((_BLK, DH), lambda i: (i, 0)),
        pl.BlockSpec((_BLK, DH), lambda i: (i + _GRID, 0)),
        pl.BlockSpec((_BLK, DH), lambda i: (i, 0)),
        pl.BlockSpec((_BLK, DH), lambda i: (i + _GRID, 0)),
        pl.BlockSpec((_BLK, 1), lambda i: (i, 0)),
        pl.BlockSpec((DH, DO3), lambda i: (0, 0)),
        pl.BlockSpec((DH, DO3), lambda i: (0, 0)),
        pl.BlockSpec((1, DO3), lambda i: (0, 0)),
        pl.BlockSpec((DO3, 3), lambda i: (0, 0)),
        pl.BlockSpec((1, 3), lambda i: (0, 0)),
    ],
    out_specs=pl.BlockSpec((_BLK, D), lambda i: (i, 0)),
    out_shape=jax.ShapeDtypeStruct((N, D), jnp.float32),
)


def kernel(x, edge_index, W1, b1, W2, b2, W3, b3, Wa, ba):
    src = edge_index[0]
    dst = edge_index[1]
    pad = jnp.full((E_PAD - E,), N, dtype=jnp.int32)
    srcp = jnp.concatenate([src, pad])
    dstp = jnp.concatenate([dst, pad]).reshape(NW * CPW, CHUNK)

    ones16 = jnp.ones((CHUNK, 16), jnp.float32)
    zeros16 = jnp.zeros((RCHUNK, 16), jnp.float32)
    dego = _deg_kernel(dstp, ones16, zeros16)
    deg = dego[:NP, 0] + dego[NP:, 0] + 1.0
    dinv = lax.rsqrt(deg)

    xd = x * dinv[:N, None]
    ysplit = (jnp.zeros((NC * NP, DH), jnp.float32)
              .at[:N].set(xd[:, :DH]).at[NP:NP + N].set(xd[:, DH:]))
    srcp2 = srcp.reshape(NS * CPS, CHUNK)
    dstp2 = dstp.reshape(NS * CPS, CHUNK)
    zeros64 = jnp.zeros((RCHUNK, DH), jnp.float32)
    ybf = (ysplit.reshape(-1, DH // 32, 2, 16).transpose(0, 1, 3, 2)
           .reshape(-1, DH).astype(jnp.bfloat16))
    ypk = lax.bitcast_convert_type(ybf.reshape(-1, DH // 2, 2), jnp.int32)
    z = _scatter_kernel(ypk, srcp2, dstp2, zeros64)

    Wcat = jnp.concatenate([W1, W2, W3], axis=1)
    bcat = jnp.concatenate([b1, b2, b3]).reshape(1, DO3)
    return _finish(z, z, ysplit, ysplit, dinv[:, None], Wcat[:DH], Wcat[DH:],
                   bcat, Wa, ba.reshape(1, 3))
